# baseline (speedup 1.0000x reference)
import functools

import jax
import jax.numpy as jnp
from jax import lax
from jax.experimental import pallas as pl
from jax.experimental.pallas import tpu as pltpu
from jax.experimental.pallas import tpu_sc as plsc

VOCAB = 1000000
D = 128
C = 2
B = 4096
S = 200

NC = 2
NS = 16
NW = NC * NS
RW = B // NW
NG = RW // 16
S0 = 104
S1 = S - S0
NV = D // 16

_mesh = plsc.VectorSubcoreMesh(core_axis_name="c", subcore_axis_name="s")


@functools.partial(
    pl.kernel,
    out_type=jax.ShapeDtypeStruct((C * B,), jnp.float32),
    mesh=_mesh,
    compiler_params=pltpu.CompilerParams(needs_layout_passes=False),
    scratch_types=[
        pltpu.VMEM((S0,), jnp.int32),
        pltpu.VMEM((S1,), jnp.int32),
        pltpu.VMEM((S, D), jnp.float32),
        pltpu.VMEM((C, D), jnp.float32),
        pltpu.VMEM((16,), jnp.float32),
        pltpu.VMEM((256,), jnp.float32),
        pltpu.VMEM((256,), jnp.float32),
        pltpu.VMEM((C, RW), jnp.float32),
        pltpu.SemaphoreType.DMA,
    ],
)
def _fasttext_sc(x_hbm, emb_hbm, w_hbm, b_hbm, out_hbm,
                 idx_a, idx_b, rows, w_v, b_v, pbuf0, pbuf1, out_v, sem):
    wid = lax.axis_index("s") * NC + lax.axis_index("c")
    base = wid * RW
    pltpu.sync_copy(w_hbm, w_v)
    pltpu.sync_copy(b_hbm, b_v)
    inv_s = jnp.float32(1.0 / S)
    bvec = b_v[...]
    lane = lax.iota(jnp.int32, 16)
    w_regs = [[w_v[c, pl.ds(j * 16, 16)] for j in range(NV)] for c in range(C)]

    def group_body(g, carry):
        def row_body(k, carry2):
            r = base + g * 16 + k
            pltpu.sync_copy(x_hbm.at[pl.ds(r * S, S0)], idx_a)
            pltpu.sync_copy(x_hbm.at[pl.ds(r * S + S0, S1)], idx_b)
            cp_a = pltpu.async_copy(
                emb_hbm.at[idx_a], rows.at[pl.ds(0, S0)], sem)
            cp_b = pltpu.async_copy(
                emb_hbm.at[idx_b], rows.at[pl.ds(S0, S1)], sem)
            cp_a.wait()
            cp_b.wait()

            def red(s, acc):
                return tuple(
                    acc[j] + rows[s, pl.ds(j * 16, 16)] for j in range(NV))

            acc = lax.fori_loop(
                0, S, red,
                tuple(jnp.zeros((16,), jnp.float32) for _ in range(NV)),
                unroll=2)
            for c, pb in ((0, pbuf0), (1, pbuf1)):
                pv = acc[0] * w_regs[c][0]
                for j in range(1, NV):
                    pv = pv + acc[j] * w_regs[c][j]
                pb[pl.ds(k * 16, 16)] = pv
            return carry2

        lax.fori_loop(0, 16, row_body, 0)
        for c, pb in ((0, pbuf0), (1, pbuf1)):
            tot = jnp.zeros((16,), jnp.float32)
            for l in range(16):
                tot = tot + plsc.load_gather(pb, [lane * 16 + l])
            out_v[c, pl.ds(g * 16, 16)] = tot * inv_s + bvec[c]
        return carry

    lax.fori_loop(0, NG, group_body, 0)
    pltpu.sync_copy(out_v.at[0], out_hbm.at[pl.ds(base, RW)])
    pltpu.sync_copy(out_v.at[1], out_hbm.at[pl.ds(B + base, RW)])


def kernel(x, emb, W, b):
    b_pad = jnp.pad(b.astype(jnp.float32), (0, 16 - C))
    out_cm = _fasttext_sc(x.astype(jnp.int32).reshape(-1), emb, W, b_pad)
    return out_cm.reshape(C, B).T


# speedup vs baseline: 2.2453x; 2.2453x over previous
import functools

import jax
import jax.numpy as jnp
from jax import lax
from jax.experimental import pallas as pl
from jax.experimental.pallas import tpu as pltpu
from jax.experimental.pallas import tpu_sc as plsc

VOCAB = 1000000
D = 128
C = 2
B = 4096
S = 200

NC = 2
NS = 16
NW = NC * NS
RW = B // NW
NG = RW // 16
S0 = 104
S1 = S - S0
NV = D // 16

_mesh = plsc.VectorSubcoreMesh(core_axis_name="c", subcore_axis_name="s")


@functools.partial(
    pl.kernel,
    out_type=jax.ShapeDtypeStruct((C * B,), jnp.float32),
    mesh=_mesh,
    compiler_params=pltpu.CompilerParams(needs_layout_passes=False),
    scratch_types=[
        pltpu.VMEM((RW * S,), jnp.int32),
        pltpu.VMEM((S, D), jnp.float32),
        pltpu.VMEM((S, D), jnp.float32),
        pltpu.VMEM((C, D), jnp.float32),
        pltpu.VMEM((16,), jnp.float32),
        pltpu.VMEM((256,), jnp.float32),
        pltpu.VMEM((256,), jnp.float32),
        pltpu.VMEM((C, RW), jnp.float32),
        pltpu.SemaphoreType.DMA,
        pltpu.SemaphoreType.DMA,
    ],
)
def _fasttext_sc(x_hbm, emb_hbm, w_hbm, b_hbm, out_hbm,
                 idx_all, rows0, rows1, w_v, b_v, pbuf0, pbuf1, out_v,
                 sem0, sem1):
    wid = lax.axis_index("s") * NC + lax.axis_index("c")
    base = wid * RW
    pltpu.sync_copy(x_hbm.at[pl.ds(base * S, RW * S)], idx_all)
    pltpu.sync_copy(w_hbm, w_v)
    pltpu.sync_copy(b_hbm, b_v)
    inv_s = jnp.float32(1.0 / S)
    bvec = b_v[...]
    lane = lax.iota(jnp.int32, 16)
    w_regs = [[w_v[c, pl.ds(j * 16, 16)] for j in range(NV)] for c in range(C)]

    def fire(i, rows_buf, sem):
        off = jnp.minimum(i, RW - 1) * S
        pltpu.async_copy(
            emb_hbm.at[idx_all.at[pl.ds(off, S0)]],
            rows_buf.at[pl.ds(0, S0)], sem)
        pltpu.async_copy(
            emb_hbm.at[idx_all.at[pl.ds(off + S0, S1)]],
            rows_buf.at[pl.ds(S0, S1)], sem)

    def drain(rows_buf, sem):
        pltpu.make_async_copy(
            emb_hbm.at[idx_all.at[pl.ds(0, S0)]],
            rows_buf.at[pl.ds(0, S0)], sem).wait()
        pltpu.make_async_copy(
            emb_hbm.at[idx_all.at[pl.ds(0, S1)]],
            rows_buf.at[pl.ds(S0, S1)], sem).wait()

    def reduce_project(rows_buf, k):
        def red(s, acc):
            return tuple(
                acc[j] + rows_buf[s, pl.ds(j * 16, 16)] for j in range(NV))

        acc = lax.fori_loop(
            0, S, red,
            tuple(jnp.zeros((16,), jnp.float32) for _ in range(NV)),
            unroll=8)
        for c, pb in ((0, pbuf0), (1, pbuf1)):
            pv = acc[0] * w_regs[c][0]
            for j in range(1, NV):
                pv = pv + acc[j] * w_regs[c][j]
            pb[pl.ds(k * 16, 16)] = pv

    fire(0, rows0, sem0)

    def group_body(g, carry):
        def pair_body(t, carry2):
            i0 = g * 16 + 2 * t
            fire(i0 + 1, rows1, sem1)
            drain(rows0, sem0)
            reduce_project(rows0, 2 * t)
            fire(i0 + 2, rows0, sem0)
            drain(rows1, sem1)
            reduce_project(rows1, 2 * t + 1)
            return carry2

        lax.fori_loop(0, 8, pair_body, 0)
        for c, pb in ((0, pbuf0), (1, pbuf1)):
            tot = jnp.zeros((16,), jnp.float32)
            for l in range(16):
                tot = tot + plsc.load_gather(pb, [lane * 16 + l])
            out_v[c, pl.ds(g * 16, 16)] = tot * inv_s + bvec[c]
        return carry

    lax.fori_loop(0, NG, group_body, 0)
    drain(rows0, sem0)
    pltpu.sync_copy(out_v.at[0], out_hbm.at[pl.ds(base, RW)])
    pltpu.sync_copy(out_v.at[1], out_hbm.at[pl.ds(B + base, RW)])


def kernel(x, emb, W, b):
    b_pad = jnp.pad(b.astype(jnp.float32), (0, 16 - C))
    out_cm = _fasttext_sc(x.astype(jnp.int32).reshape(-1), emb, W, b_pad)
    return out_cm.reshape(C, B).T
